# Initial kernel scaffold; baseline (speedup 1.0000x reference)
import jax
import jax.numpy as jnp
from jax.experimental import pallas as pl


def kernel(x, clusters_edge_index, B, clusters, Wg1, bg1, Wg2, bg2, W1, W2, W3, W4, W5):
    raise NotImplementedError("write your pallas kernel here")


# speedup vs baseline: 17.5175x; 17.5175x over previous
import functools

import jax
import jax.numpy as jnp
from jax import lax
from jax.experimental import pallas as pl
from jax.experimental.pallas import tpu as pltpu
from jax.experimental.pallas import tpu_sc as plsc

N = 50000
E = 800000
H = 64
NC = 100
NB = 4096
FIN = 24
GH = 96

NCORES = 2
NSUB = 16
NW = NCORES * NSUB

NPAD = 50176
EPAD = 802816
GARBAGE_ROW = N
CBINS = 16384


_DEG_PER_TILE = EPAD // NW
_DEG_CHUNK = 1568
_CL_PER_TILE = NPAD // NW


def _hist_body(dst_hbm, nodei_hbm, clusi_hbm, degT_hbm, chT_hbm,
               deg_local, ch_local, dbuf, nbuf, cbuf):
  c = lax.axis_index("c")
  s = lax.axis_index("s")
  w = c * NSUB + s

  z16 = jnp.zeros((16,), jnp.float32)

  def zero_deg(i, _):
    deg_local[pl.ds(i * 16, 16)] = z16
    return _
  lax.fori_loop(0, NPAD // 16, zero_deg, 0)

  def zero_ch(i, _):
    ch_local[pl.ds(i * 16, 16)] = z16
    return _
  lax.fori_loop(0, CBINS // 16, zero_ch, 0)

  probe = jnp.zeros((16,), jnp.int32)
  pcnt, _ = plsc.scan_count(probe)
  corr = (16 - jnp.max(pcnt)).astype(jnp.float32)

  def deg_chunk(k, _):
    base = w * _DEG_PER_TILE + k * _DEG_CHUNK
    pltpu.sync_copy(dst_hbm.at[pl.ds(base, _DEG_CHUNK)], dbuf)

    def vec(j, _):
      v = dbuf[pl.ds(j * 16, 16)]
      cnt, last = plsc.scan_count(v)
      plsc.addupdate_scatter(deg_local, [v], cnt.astype(jnp.float32) + corr,
                             mask=last)
      return _
    lax.fori_loop(0, _DEG_CHUNK // 16, vec, 0)
    return _
  lax.fori_loop(0, _DEG_PER_TILE // _DEG_CHUNK, deg_chunk, 0)

  cbase = w * _CL_PER_TILE
  pltpu.sync_copy(nodei_hbm.at[pl.ds(cbase, _CL_PER_TILE)], nbuf)
  pltpu.sync_copy(clusi_hbm.at[pl.ds(cbase, _CL_PER_TILE)], cbuf)

  def cl_vec(j, _):
    nv = nbuf[pl.ds(j * 16, 16)]
    cv = cbuf[pl.ds(j * 16, 16)]
    comb = cv * 128 + nv
    cnt, last = plsc.scan_count(comb)
    plsc.addupdate_scatter(ch_local, [comb], cnt.astype(jnp.float32) + corr,
                           mask=last)
    return _
  lax.fori_loop(0, _CL_PER_TILE // 16, cl_vec, 0)

  pltpu.sync_copy(deg_local, degT_hbm.at[w])
  pltpu.sync_copy(ch_local, chT_hbm.at[w])


def _make_hist_kernel():
  mesh = plsc.VectorSubcoreMesh(core_axis_name="c", subcore_axis_name="s",
                                num_cores=NCORES, num_subcores=NSUB)
  return pl.kernel(
      _hist_body,
      compiler_params=pltpu.CompilerParams(needs_layout_passes=False),
      out_type=[
          jax.ShapeDtypeStruct((NW, NPAD), jnp.float32),
          jax.ShapeDtypeStruct((NW, CBINS), jnp.float32),
      ],
      mesh=mesh,
      scratch_types=[
          pltpu.VMEM((NPAD,), jnp.float32),
          pltpu.VMEM((CBINS,), jnp.float32),
          pltpu.VMEM((_DEG_CHUNK,), jnp.int32),
          pltpu.VMEM((_CL_PER_TILE,), jnp.int32),
          pltpu.VMEM((_CL_PER_TILE,), jnp.int32),
      ],
  )


_ACC_ROWS = NPAD
_ZROWS = 784
_GRP = 128
_BLK = 4


def _prop_body(src_hbm, dst2d_hbm, table_hbm, out_hbm,
               sidx, didx, rows, zbuf, acc, sem,
               *, edges_per_tile, dim_split):
  c = lax.axis_index("c")
  s = lax.axis_index("s")

  z16 = jnp.zeros((16,), jnp.float32)

  def zero_row(i, _):
    zbuf[i, pl.ds(0, 16)] = z16
    zbuf[i, pl.ds(16, 16)] = z16
    return _
  lax.fori_loop(0, _ZROWS, zero_row, 0)
  for q in range(4):
    pltpu.sync_copy(zbuf, acc.at[pl.ds(s * 3136 + q * _ZROWS, _ZROWS)])
  plsc.subcore_barrier()

  if dim_split:
    edge_base = s * edges_per_tile
    off = c * N
  else:
    edge_base = (c * NSUB + s) * edges_per_tile
    off = None

  nblocks = edges_per_tile // (_BLK * _GRP)

  def block(b, _):
    base = edge_base + b * (_BLK * _GRP)
    pltpu.sync_copy(src_hbm.at[pl.ds(base // _GRP, _BLK)], sidx)
    pltpu.sync_copy(dst2d_hbm.at[pl.ds(base // _GRP, _BLK)], didx)
    if off is not None:
      for j in range(_BLK):
        for k in range(_GRP // 16):
          sidx[j, pl.ds(k * 16, 16)] = sidx[j, pl.ds(k * 16, 16)] + off
    for j in range(_BLK):
      pltpu.async_copy(table_hbm.at[sidx.at[j]], rows, sem).wait()
      pltpu.sync_copy(rows, acc.at[didx.at[j]], add=True)
    return _
  lax.fori_loop(0, nblocks, block, 0)

  plsc.subcore_barrier()
  pltpu.sync_copy(acc.at[pl.ds(s * 3125, 3125)],
                  out_hbm.at[c, pl.ds(s * 3125, 3125)])


def _make_prop_kernel(edges_per_tile, dim_split):
  mesh = plsc.VectorSubcoreMesh(core_axis_name="c", subcore_axis_name="s",
                                num_cores=NCORES, num_subcores=NSUB)
  body = functools.partial(_prop_body, edges_per_tile=edges_per_tile,
                           dim_split=dim_split)
  return pl.kernel(
      body,
      compiler_params=pltpu.CompilerParams(needs_layout_passes=False,
                                           use_tc_tiling_on_sc=False),
      out_type=jax.ShapeDtypeStruct((NCORES, N, 32), jnp.float32),
      mesh=mesh,
      scratch_types=[
          pltpu.VMEM((_BLK, _GRP), jnp.int32),
          pltpu.VMEM((_BLK, _GRP), jnp.int32),
          pltpu.VMEM((_GRP, 32), jnp.float32),
          pltpu.VMEM((_ZROWS, 32), jnp.float32),
          pltpu.VMEM_SHARED((_ACC_ROWS, 32), jnp.float32),
          pltpu.SemaphoreType.DMA,
      ],
  )


def _dinv_body(degT_ref, dinv_ref):
  d = jnp.sum(degT_ref[...], axis=0, keepdims=True) + 1.0
  dinv_ref[...] = lax.rsqrt(d)


def _xs_body(x_ref, dinv_ref, xs_ref):
  xb = x_ref[...] * dinv_ref[...]
  xs_ref[...] = jnp.concatenate(
      [xb, jnp.zeros((xb.shape[0], 32 - FIN), jnp.float32)], axis=1)


def _mid_body(s1_ref, xs_ref, dinv_ref, wg1_ref, bg1_ref, wg2_ref, y2_ref):
  dinv = dinv_ref[...]
  z1 = (s1_ref[0] + s1_ref[1] + xs_ref[...]) * dinv
  h1 = jnp.maximum(
      lax.dot_general(z1, wg1_ref[...], (((1,), (1,)), ((), ())),
                      preferred_element_type=jnp.float32) + bg1_ref[...], 0.0)
  hw2 = lax.dot_general(h1, wg2_ref[...], (((1,), (1,)), ((), ())),
                        preferred_element_type=jnp.float32)
  y = hw2 * dinv
  y2_ref[0] = y[:, :32]
  y2_ref[1] = y[:, 32:]


def _softmax_body(s2_ref, y2_ref, dinv_ref, bg2_ref, colsum_ref, nvec_ref):
  i = pl.program_id(0)
  za = s2_ref[0] + y2_ref[0]
  zb = s2_ref[1] + y2_ref[1]
  h2 = jnp.concatenate([za, zb], axis=1) * dinv_ref[...] + bg2_ref[...]
  m = jnp.max(h2, axis=1, keepdims=True)
  sh = h2 - m
  nv = sh - jnp.log(jnp.sum(jnp.exp(sh), axis=1, keepdims=True))

  @pl.when(i == 0)
  def _():
    colsum_ref[...] = jnp.zeros_like(colsum_ref)
    nvec_ref[...] = nv[:128]

  colsum_ref[...] += jnp.sum(nv, axis=0, keepdims=True)


def _tail_body(colsum_ref, nvec_ref, chT_ref, b_ref,
               w2_ref, w4_ref, w5_ref, w3_ref, w1_ref, out_ref):
  f32 = jnp.float32
  dn = (((1,), (1,)), ((), ()))

  ch = jnp.sum(chT_ref[...], axis=0)
  nvec = nvec_ref[...]
  gv = lax.dot_general(colsum_ref[...], w2_ref[...], dn,
                       preferred_element_type=f32)
  g0 = jnp.sum(jnp.maximum(gv, 0.0) * w1_ref[:, :H], axis=1,
               keepdims=True)
  cvec = lax.dot_general(ch, nvec, (((1,), (0,)), ((), ())),
                         preferred_element_type=f32)
  cv = lax.dot_general(cvec, w5_ref[...], dn,
                       preferred_element_type=f32)
  nv = lax.dot_general(nvec, w4_ref[...], dn,
                       preferred_element_type=f32)

  bblk = b_ref[...]
  bm = bblk.shape[0]
  iota = lax.broadcasted_iota(jnp.int32, (bm, 128), 1)
  temp = jnp.zeros((bm, H), f32)
  for k, tbl in ((0, cv), (1, nv), (2, cv), (3, nv)):
    oh = (iota == bblk[:, k:k + 1]).astype(f32)
    sel = lax.dot_general(oh, tbl, (((1,), (0,)), ((), ())),
                          preferred_element_type=f32)
    temp = temp + lax.dot_general(
        jnp.maximum(sel, 0.0), w3_ref[:, H * k:H * (k + 1)], dn,
        preferred_element_type=f32)
  q2 = jnp.sum(jnp.maximum(temp, 0.0) * w1_ref[:, H:], axis=1,
               keepdims=True)
  out_ref[...] = q2 + g0


def kernel(x, clusters_edge_index, B, clusters, Wg1, bg1, Wg2, bg2,
           W1, W2, W3, W4, W5):
  f32 = jnp.float32
  src = clusters_edge_index[0].astype(jnp.int32)
  dst = clusters_edge_index[1].astype(jnp.int32)
  src_p = jnp.pad(src, (0, EPAD - E), constant_values=0)
  dst_p = jnp.pad(dst, (0, EPAD - E), constant_values=GARBAGE_ROW)
  src2d = src_p.reshape(EPAD // _GRP, _GRP)
  dst2d = dst_p.reshape(EPAD // _GRP, _GRP)

  nodei = jnp.pad(clusters[:, 0].astype(jnp.int32), (0, NPAD - N),
                  constant_values=127)
  clusi = jnp.pad(clusters[:, 1].astype(jnp.int32), (0, NPAD - N),
                  constant_values=127)

  degT, chT = _make_hist_kernel()(dst_p, nodei, clusi)

  dinv_row = pl.pallas_call(
      _dinv_body,
      grid=(NPAD // 128,),
      in_specs=[pl.BlockSpec((NW, 128), lambda i: (0, i))],
      out_specs=pl.BlockSpec((1, 128), lambda i: (0, i)),
      out_shape=jax.ShapeDtypeStruct((1, NPAD), f32),
  )(degT)
  dinv_col = dinv_row.reshape(NPAD, 1)

  xs = pl.pallas_call(
      _xs_body,
      grid=(NPAD // 512,),
      in_specs=[pl.BlockSpec((512, FIN), lambda i: (i, 0)),
                pl.BlockSpec((512, 1), lambda i: (i, 0))],
      out_specs=pl.BlockSpec((512, 32), lambda i: (i, 0)),
      out_shape=jax.ShapeDtypeStruct((NPAD, 32), f32),
  )(x.astype(f32), dinv_col)

  s1 = _make_prop_kernel(EPAD // NW, dim_split=False)(
      src2d, dst2d, xs)

  wg1p = jnp.pad(Wg1.astype(f32), ((0, 0), (0, 32 - FIN)))
  bm = 1000
  y2 = pl.pallas_call(
      _mid_body,
      grid=(N // bm,),
      in_specs=[pl.BlockSpec((2, bm, 32), lambda i: (0, i, 0)),
                pl.BlockSpec((bm, 32), lambda i: (i, 0)),
                pl.BlockSpec((bm, 1), lambda i: (i, 0)),
                pl.BlockSpec((GH, 32), lambda i: (0, 0)),
                pl.BlockSpec((1, GH), lambda i: (0, 0)),
                pl.BlockSpec((H, GH), lambda i: (0, 0))],
      out_specs=pl.BlockSpec((2, bm, 32), lambda i: (0, i, 0)),
      out_shape=jax.ShapeDtypeStruct((2, N, 32), f32),
  )(s1, xs, dinv_col, wg1p, bg1.astype(f32).reshape(1, GH), Wg2.astype(f32))

  s2 = _make_prop_kernel(EPAD // NSUB, dim_split=True)(
      src2d, dst2d, y2.reshape(2 * N, 32))

  colsum, nvec128 = pl.pallas_call(
      _softmax_body,
      grid=(N // bm,),
      in_specs=[pl.BlockSpec((2, bm, 32), lambda i: (0, i, 0)),
                pl.BlockSpec((2, bm, 32), lambda i: (0, i, 0)),
                pl.BlockSpec((bm, 1), lambda i: (i, 0)),
                pl.BlockSpec((1, H), lambda i: (0, 0))],
      out_specs=[pl.BlockSpec((1, H), lambda i: (0, 0)),
                 pl.BlockSpec((128, H), lambda i: (0, 0))],
      out_shape=[jax.ShapeDtypeStruct((1, H), f32),
                 jax.ShapeDtypeStruct((128, H), f32)],
  )(s2, y2, dinv_col, bg2.astype(f32).reshape(1, H))

  bq = 512
  out = pl.pallas_call(
      _tail_body,
      grid=(NB // bq,),
      in_specs=[pl.BlockSpec((1, H), lambda i: (0, 0)),
                pl.BlockSpec((128, H), lambda i: (0, 0)),
                pl.BlockSpec((NW, 128, 128), lambda i: (0, 0, 0)),
                pl.BlockSpec((bq, 4), lambda i: (i, 0)),
                pl.BlockSpec((H, H), lambda i: (0, 0)),
                pl.BlockSpec((H, H), lambda i: (0, 0)),
                pl.BlockSpec((H, H), lambda i: (0, 0)),
                pl.BlockSpec((H, 4 * H), lambda i: (0, 0)),
                pl.BlockSpec((1, 2 * H), lambda i: (0, 0))],
      out_specs=pl.BlockSpec((bq, 1), lambda i: (i, 0)),
      out_shape=jax.ShapeDtypeStruct((NB, 1), f32),
  )(colsum, nvec128, chT.reshape(NW, 128, 128), B.astype(jnp.int32),
    W2.astype(f32), W4.astype(f32), W5.astype(f32), W3.astype(f32),
    W1.astype(f32))
  return out
